# baseline (speedup 1.0000x reference)
import functools

import jax
import jax.numpy as jnp
from jax.experimental import pallas as pl
from jax.experimental.pallas import tpu as pltpu

EPS_BN = 1e-5
MB = 512
MB1 = 2048


def _body(B, N, M, nmb, np0,
          xf_ref, xoldf_ref, wup_ref, wlat_ref, pt_ref, pold_ref, gb_ref,
          y_ref, h_ref, zlat_ref, stats_ref, aff_ref):
    i = pl.program_id(0)
    n_up = float(B * N)
    n_lat = float(B * M)

    @pl.when(i == 0)
    def _up():
        zup = jax.lax.dot_general(
            xf_ref[...], wup_ref[...], (((1,), (1,)), ((), ())),
            preferred_element_type=jnp.float32)
        h_ref[...] = zup
        stats_ref[0:1, :] = jnp.sum(zup, axis=0, keepdims=True)
        stats_ref[1:2, :] = jnp.sum(zup * zup, axis=0, keepdims=True)
        stats_ref[2:4, :] = jnp.zeros_like(stats_ref[2:4, :])

    @pl.when(i < np0)
    def _lat_mm():
        zlat = jax.lax.dot_general(
            xoldf_ref[...], wlat_ref[...], (((1,), (1,)), ((), ())),
            preferred_element_type=jnp.float32)
        zlat_ref[pl.ds(i * MB1, MB1), :] = zlat
        stats_ref[2:3, :] += jnp.sum(zlat, axis=0, keepdims=True)
        stats_ref[3:4, :] += jnp.sum(zlat * zlat, axis=0, keepdims=True)

    @pl.when(i == np0)
    def _affine():
        mean_up = stats_ref[0:1, :] / n_up
        var_up = jnp.maximum(stats_ref[1:2, :] / n_up - mean_up * mean_up, 0.0)
        scale_up = gb_ref[0:1, :] * jax.lax.rsqrt(var_up + EPS_BN)
        shift_up = gb_ref[1:2, :] - mean_up * scale_up
        mean_lat = stats_ref[2:3, :] / n_lat
        var_lat = jnp.maximum(
            stats_ref[3:4, :] / n_lat - mean_lat * mean_lat, 0.0)
        scale_lat = gb_ref[2:3, :] * jax.lax.rsqrt(var_lat + EPS_BN)
        aff_ref[2:3, :] = scale_lat
        aff_ref[3:4, :] = gb_ref[3:4, :] - mean_lat * scale_lat
        h_ref[...] = jnp.maximum(h_ref[...] * scale_up + shift_up, 0.0)

    @pl.when(i >= np0)
    def _knn():
        j = i - np0
        b = j // nmb

        pold = pold_ref[0]
        pt = pt_ref[0]
        d0 = pold[:, 0:1] - pt[0:1, :]
        d1 = pold[:, 1:2] - pt[1:2, :]
        d2c = pold[:, 2:3] - pt[2:3, :]
        d = d0 * d0 + d1 * d1 + d2c * d2c

        inf = jnp.float32(jnp.inf)
        m0 = jnp.min(d, axis=1, keepdims=True)
        s0 = d == m0
        dm = jnp.where(s0, inf, d)
        m1 = jnp.min(dm, axis=1, keepdims=True)
        s1 = dm == m1
        dm = jnp.where(s1, inf, dm)
        m2 = jnp.min(dm, axis=1, keepdims=True)
        s2 = dm == m2

        w0 = 1.0 / jnp.maximum(m0, 1e-16)
        w1 = 1.0 / jnp.maximum(m1, 1e-16)
        w2 = 1.0 / jnp.maximum(m2, 1e-16)
        ws = w0 + w1 + w2
        zero = jnp.zeros_like(d)
        a = jnp.where(s0, w0 / ws,
                      jnp.where(s1, w1 / ws, jnp.where(s2, w2 / ws, zero)))

        h_b = h_ref[pl.ds(b * N, N), :]
        interp = jax.lax.dot_general(
            a, h_b, (((1,), (0,)), ((), ())),
            preferred_element_type=jnp.float32)
        zlat_b = zlat_ref[pl.ds(j * MB, MB), :]
        lat = jnp.maximum(zlat_b * aff_ref[2:3, :] + aff_ref[3:4, :], 0.0)
        y_ref[0] = interp + lat


@functools.partial(jax.jit, static_argnames=())
def kernel(x, p, x_old, p_old, W_up, gamma_up, beta_up,
           W_lat, gamma_lat, beta_lat):
    B, N, Cin = x.shape
    M = p_old.shape[1]
    Cout = W_up.shape[0]
    nmb = M // MB
    np0 = (B * M) // MB1
    nsteps = np0 + B * nmb

    xf = x.reshape(B * N, Cin)
    xoldf = x_old.reshape(B * M, Cout)
    gb = jnp.stack([gamma_up, beta_up, gamma_lat, beta_lat], axis=0)
    p_t = jnp.transpose(p, (0, 2, 1))

    def _b_of(i):
        return jnp.where(i < np0, 0, (i - np0) // nmb)

    def _m_of(i):
        return jnp.where(i < np0, 0, (i - np0) % nmb)

    y = pl.pallas_call(
        functools.partial(_body, B, N, M, nmb, np0),
        grid=(nsteps,),
        in_specs=[
            pl.BlockSpec((B * N, Cin), lambda i: (0, 0)),
            pl.BlockSpec((MB1, Cout), lambda i: (jnp.where(i < np0, i, 0), 0)),
            pl.BlockSpec((Cout, Cin), lambda i: (0, 0)),
            pl.BlockSpec((Cout, Cout), lambda i: (0, 0)),
            pl.BlockSpec((1, 3, N), lambda i: (_b_of(i), 0, 0)),
            pl.BlockSpec((1, MB, 3), lambda i: (_b_of(i), _m_of(i), 0)),
            pl.BlockSpec((4, Cout), lambda i: (0, 0)),
        ],
        out_specs=pl.BlockSpec(
            (1, MB, Cout), lambda i: (_b_of(i), _m_of(i), 0)),
        out_shape=jax.ShapeDtypeStruct((B, M, Cout), jnp.float32),
        scratch_shapes=[
            pltpu.VMEM((B * N, Cout), jnp.float32),
            pltpu.VMEM((B * M, Cout), jnp.float32),
            pltpu.VMEM((4, Cout), jnp.float32),
            pltpu.VMEM((4, Cout), jnp.float32),
        ],
        compiler_params=pltpu.CompilerParams(
            dimension_semantics=("arbitrary",)),
    )(xf, xoldf, W_up, W_lat, p_t, p_old, gb)

    return (y, p_old)


# speedup vs baseline: 1.0761x; 1.0761x over previous
import functools

import jax
import jax.numpy as jnp
from jax.experimental import pallas as pl
from jax.experimental.pallas import tpu as pltpu

EPS_BN = 1e-5
MB = 1024
MB1 = 2048


def _body(B, N, M, nmb, np0,
          xf_ref, xoldf_ref, wup_ref, wlat_ref, pt_ref, pold_ref, gb_ref,
          y_ref, h_ref, zlat_ref, stats_ref, aff_ref):
    i = pl.program_id(0)
    n_up = float(B * N)
    n_lat = float(B * M)

    @pl.when(i == 0)
    def _up():
        zup = jax.lax.dot_general(
            xf_ref[...], wup_ref[...], (((1,), (1,)), ((), ())),
            preferred_element_type=jnp.float32)
        h_ref[...] = zup
        stats_ref[0:1, :] = jnp.sum(zup, axis=0, keepdims=True)
        stats_ref[1:2, :] = jnp.sum(zup * zup, axis=0, keepdims=True)
        stats_ref[2:4, :] = jnp.zeros_like(stats_ref[2:4, :])

    @pl.when(i < np0)
    def _lat_mm():
        zlat = jax.lax.dot_general(
            xoldf_ref[...], wlat_ref[...], (((1,), (1,)), ((), ())),
            preferred_element_type=jnp.float32)
        zlat_ref[pl.ds(i * MB1, MB1), :] = zlat
        stats_ref[2:3, :] += jnp.sum(zlat, axis=0, keepdims=True)
        stats_ref[3:4, :] += jnp.sum(zlat * zlat, axis=0, keepdims=True)

    @pl.when(i == np0)
    def _affine():
        mean_up = stats_ref[0:1, :] / n_up
        var_up = jnp.maximum(stats_ref[1:2, :] / n_up - mean_up * mean_up, 0.0)
        scale_up = gb_ref[0:1, :] * jax.lax.rsqrt(var_up + EPS_BN)
        shift_up = gb_ref[1:2, :] - mean_up * scale_up
        mean_lat = stats_ref[2:3, :] / n_lat
        var_lat = jnp.maximum(
            stats_ref[3:4, :] / n_lat - mean_lat * mean_lat, 0.0)
        scale_lat = gb_ref[2:3, :] * jax.lax.rsqrt(var_lat + EPS_BN)
        aff_ref[2:3, :] = scale_lat
        aff_ref[3:4, :] = gb_ref[3:4, :] - mean_lat * scale_lat
        h_ref[...] = jnp.maximum(h_ref[...] * scale_up + shift_up, 0.0)

    @pl.when(i >= np0)
    def _knn():
        j = i - np0
        b = j // nmb

        pold = pold_ref[0]
        pt = pt_ref[0]
        d0 = pold[:, 0:1] - pt[0:1, :]
        d1 = pold[:, 1:2] - pt[1:2, :]
        d2c = pold[:, 2:3] - pt[2:3, :]
        d = d0 * d0 + d1 * d1 + d2c * d2c

        inf = jnp.float32(jnp.inf)
        m0 = jnp.min(d, axis=1, keepdims=True)
        s0 = d == m0
        dm = jnp.where(s0, inf, d)
        m1 = jnp.min(dm, axis=1, keepdims=True)
        s1 = dm == m1
        dm = jnp.where(s1, inf, dm)
        m2 = jnp.min(dm, axis=1, keepdims=True)
        s2 = dm == m2

        w0 = 1.0 / jnp.maximum(m0, 1e-16)
        w1 = 1.0 / jnp.maximum(m1, 1e-16)
        w2 = 1.0 / jnp.maximum(m2, 1e-16)
        ws = w0 + w1 + w2
        zero = jnp.zeros_like(d)
        a = jnp.where(s0, w0 / ws,
                      jnp.where(s1, w1 / ws, jnp.where(s2, w2 / ws, zero)))

        h_b = h_ref[pl.ds(b * N, N), :]
        interp = jax.lax.dot_general(
            a, h_b, (((1,), (0,)), ((), ())),
            preferred_element_type=jnp.float32)
        zlat_b = zlat_ref[pl.ds(j * MB, MB), :]
        lat = jnp.maximum(zlat_b * aff_ref[2:3, :] + aff_ref[3:4, :], 0.0)
        y_ref[0] = interp + lat


@functools.partial(jax.jit, static_argnames=())
def kernel(x, p, x_old, p_old, W_up, gamma_up, beta_up,
           W_lat, gamma_lat, beta_lat):
    B, N, Cin = x.shape
    M = p_old.shape[1]
    Cout = W_up.shape[0]
    nmb = M // MB
    np0 = (B * M) // MB1
    nsteps = np0 + B * nmb

    xf = x.reshape(B * N, Cin)
    xoldf = x_old.reshape(B * M, Cout)
    gb = jnp.stack([gamma_up, beta_up, gamma_lat, beta_lat], axis=0)
    p_t = jnp.transpose(p, (0, 2, 1))

    def _b_of(i):
        return jnp.where(i < np0, 0, (i - np0) // nmb)

    def _m_of(i):
        return jnp.where(i < np0, 0, (i - np0) % nmb)

    y = pl.pallas_call(
        functools.partial(_body, B, N, M, nmb, np0),
        grid=(nsteps,),
        in_specs=[
            pl.BlockSpec((B * N, Cin), lambda i: (0, 0)),
            pl.BlockSpec((MB1, Cout), lambda i: (jnp.where(i < np0, i, 0), 0)),
            pl.BlockSpec((Cout, Cin), lambda i: (0, 0)),
            pl.BlockSpec((Cout, Cout), lambda i: (0, 0)),
            pl.BlockSpec((1, 3, N), lambda i: (_b_of(i), 0, 0)),
            pl.BlockSpec((1, MB, 3), lambda i: (_b_of(i), _m_of(i), 0)),
            pl.BlockSpec((4, Cout), lambda i: (0, 0)),
        ],
        out_specs=pl.BlockSpec(
            (1, MB, Cout), lambda i: (_b_of(i), _m_of(i), 0)),
        out_shape=jax.ShapeDtypeStruct((B, M, Cout), jnp.float32),
        scratch_shapes=[
            pltpu.VMEM((B * N, Cout), jnp.float32),
            pltpu.VMEM((B * M, Cout), jnp.float32),
            pltpu.VMEM((4, Cout), jnp.float32),
            pltpu.VMEM((4, Cout), jnp.float32),
        ],
        compiler_params=pltpu.CompilerParams(
            dimension_semantics=("arbitrary",)),
    )(xf, xoldf, W_up, W_lat, p_t, p_old, gb)

    return (y, p_old)


# speedup vs baseline: 1.0846x; 1.0079x over previous
import functools

import jax
import jax.numpy as jnp
from jax.experimental import pallas as pl
from jax.experimental.pallas import tpu as pltpu

EPS_BN = 1e-5
MB = 1024
MB1 = 2048


def _body(B, N, M, nmb, np0,
          xf_ref, xoldf_ref, wup_ref, wlat_ref, pt_ref, pold_ref, gb_ref,
          y_ref, h_ref, zlat_ref, stats_ref, aff_ref):
    i = pl.program_id(0)
    n_up = float(B * N)
    n_lat = float(B * M)

    @pl.when(i == 0)
    def _up():
        zup = jax.lax.dot_general(
            xf_ref[...], wup_ref[...], (((1,), (1,)), ((), ())),
            preferred_element_type=jnp.float32)
        h_ref[...] = zup
        stats_ref[0:1, :] = jnp.sum(zup, axis=0, keepdims=True)
        stats_ref[1:2, :] = jnp.sum(zup * zup, axis=0, keepdims=True)
        stats_ref[2:4, :] = jnp.zeros_like(stats_ref[2:4, :])

    @pl.when(i < np0)
    def _lat_mm():
        zlat = jax.lax.dot_general(
            xoldf_ref[...], wlat_ref[...], (((1,), (1,)), ((), ())),
            preferred_element_type=jnp.float32)
        zlat_ref[pl.ds(i * MB1, MB1), :] = zlat
        stats_ref[2:3, :] += jnp.sum(zlat, axis=0, keepdims=True)
        stats_ref[3:4, :] += jnp.sum(zlat * zlat, axis=0, keepdims=True)

    @pl.when(i == np0)
    def _affine():
        mean_up = stats_ref[0:1, :] / n_up
        var_up = jnp.maximum(stats_ref[1:2, :] / n_up - mean_up * mean_up, 0.0)
        scale_up = gb_ref[0:1, :] * jax.lax.rsqrt(var_up + EPS_BN)
        shift_up = gb_ref[1:2, :] - mean_up * scale_up
        mean_lat = stats_ref[2:3, :] / n_lat
        var_lat = jnp.maximum(
            stats_ref[3:4, :] / n_lat - mean_lat * mean_lat, 0.0)
        scale_lat = gb_ref[2:3, :] * jax.lax.rsqrt(var_lat + EPS_BN)
        aff_ref[2:3, :] = scale_lat
        aff_ref[3:4, :] = gb_ref[3:4, :] - mean_lat * scale_lat
        h_ref[...] = jnp.maximum(h_ref[...] * scale_up + shift_up, 0.0)

    @pl.when(i >= np0)
    def _knn():
        j = i - np0
        b = j // nmb

        pold = pold_ref[0]
        pt = pt_ref[0]
        d0 = pold[:, 0:1] - pt[0:1, :]
        d1 = pold[:, 1:2] - pt[1:2, :]
        d2c = pold[:, 2:3] - pt[2:3, :]
        d = d0 * d0 + d1 * d1 + d2c * d2c

        inf = jnp.float32(jnp.inf)
        m0 = jnp.min(d, axis=1, keepdims=True)
        dm = jnp.where(d == m0, inf, d)
        m1 = jnp.min(dm, axis=1, keepdims=True)
        dm = jnp.where(dm == m1, inf, dm)
        m2 = jnp.min(dm, axis=1, keepdims=True)

        w0 = 1.0 / jnp.maximum(m0, 1e-16)
        w1 = 1.0 / jnp.maximum(m1, 1e-16)
        w2 = 1.0 / jnp.maximum(m2, 1e-16)
        ws = w0 + w1 + w2
        zero = jnp.zeros_like(d)
        a = jnp.where(d == m0, w0 / ws,
                      jnp.where(d == m1, w1 / ws,
                                jnp.where(d == m2, w2 / ws, zero)))

        h_b = h_ref[pl.ds(b * N, N), :]
        interp = jax.lax.dot_general(
            a, h_b, (((1,), (0,)), ((), ())),
            preferred_element_type=jnp.float32)
        zlat_b = zlat_ref[pl.ds(j * MB, MB), :]
        lat = jnp.maximum(zlat_b * aff_ref[2:3, :] + aff_ref[3:4, :], 0.0)
        y_ref[0] = interp + lat


@functools.partial(jax.jit, static_argnames=())
def kernel(x, p, x_old, p_old, W_up, gamma_up, beta_up,
           W_lat, gamma_lat, beta_lat):
    B, N, Cin = x.shape
    M = p_old.shape[1]
    Cout = W_up.shape[0]
    nmb = M // MB
    np0 = (B * M) // MB1
    nsteps = np0 + B * nmb

    xf = x.reshape(B * N, Cin)
    xoldf = x_old.reshape(B * M, Cout)
    gb = jnp.stack([gamma_up, beta_up, gamma_lat, beta_lat], axis=0)
    p_t = jnp.transpose(p, (0, 2, 1))

    def _b_of(i):
        return jnp.where(i < np0, 0, (i - np0) // nmb)

    def _m_of(i):
        return jnp.where(i < np0, 0, (i - np0) % nmb)

    y = pl.pallas_call(
        functools.partial(_body, B, N, M, nmb, np0),
        grid=(nsteps,),
        in_specs=[
            pl.BlockSpec((B * N, Cin), lambda i: (0, 0)),
            pl.BlockSpec((MB1, Cout), lambda i: (jnp.where(i < np0, i, 0), 0)),
            pl.BlockSpec((Cout, Cin), lambda i: (0, 0)),
            pl.BlockSpec((Cout, Cout), lambda i: (0, 0)),
            pl.BlockSpec((1, 3, N), lambda i: (_b_of(i), 0, 0)),
            pl.BlockSpec((1, MB, 3), lambda i: (_b_of(i), _m_of(i), 0)),
            pl.BlockSpec((4, Cout), lambda i: (0, 0)),
        ],
        out_specs=pl.BlockSpec(
            (1, MB, Cout), lambda i: (_b_of(i), _m_of(i), 0)),
        out_shape=jax.ShapeDtypeStruct((B, M, Cout), jnp.float32),
        scratch_shapes=[
            pltpu.VMEM((B * N, Cout), jnp.float32),
            pltpu.VMEM((B * M, Cout), jnp.float32),
            pltpu.VMEM((4, Cout), jnp.float32),
            pltpu.VMEM((4, Cout), jnp.float32),
        ],
        compiler_params=pltpu.CompilerParams(
            dimension_semantics=("arbitrary",)),
    )(xf, xoldf, W_up, W_lat, p_t, p_old, gb)

    return (y, p_old)


# speedup vs baseline: 1.0954x; 1.0099x over previous
import functools

import jax
import jax.numpy as jnp
from jax.experimental import pallas as pl
from jax.experimental.pallas import tpu as pltpu

EPS_BN = 1e-5
MB = 1024
MB1 = 4096


def _body(B, N, M, nmb, np0,
          xf_ref, xoldf_ref, wup_ref, wlat_ref, pt_ref, pold_ref, gb_ref,
          y_ref, h_ref, zlat_ref, stats_ref, aff_ref):
    i = pl.program_id(0)
    n_up = float(B * N)
    n_lat = float(B * M)

    @pl.when(i == 0)
    def _up():
        zup = jax.lax.dot_general(
            xf_ref[...], wup_ref[...], (((1,), (1,)), ((), ())),
            preferred_element_type=jnp.float32)
        h_ref[...] = zup
        stats_ref[0:1, :] = jnp.sum(zup, axis=0, keepdims=True)
        stats_ref[1:2, :] = jnp.sum(zup * zup, axis=0, keepdims=True)
        stats_ref[2:4, :] = jnp.zeros_like(stats_ref[2:4, :])

    @pl.when(i < np0)
    def _lat_mm():
        zlat = jax.lax.dot_general(
            xoldf_ref[...], wlat_ref[...], (((1,), (1,)), ((), ())),
            preferred_element_type=jnp.float32)
        zlat_ref[pl.ds(i * MB1, MB1), :] = zlat
        stats_ref[2:3, :] += jnp.sum(zlat, axis=0, keepdims=True)
        stats_ref[3:4, :] += jnp.sum(zlat * zlat, axis=0, keepdims=True)

    @pl.when(i == np0)
    def _affine():
        mean_up = stats_ref[0:1, :] / n_up
        var_up = jnp.maximum(stats_ref[1:2, :] / n_up - mean_up * mean_up, 0.0)
        scale_up = gb_ref[0:1, :] * jax.lax.rsqrt(var_up + EPS_BN)
        shift_up = gb_ref[1:2, :] - mean_up * scale_up
        mean_lat = stats_ref[2:3, :] / n_lat
        var_lat = jnp.maximum(
            stats_ref[3:4, :] / n_lat - mean_lat * mean_lat, 0.0)
        scale_lat = gb_ref[2:3, :] * jax.lax.rsqrt(var_lat + EPS_BN)
        aff_ref[2:3, :] = scale_lat
        aff_ref[3:4, :] = gb_ref[3:4, :] - mean_lat * scale_lat
        h_ref[...] = jnp.maximum(h_ref[...] * scale_up + shift_up, 0.0)

    @pl.when(i >= np0)
    def _knn():
        j = i - np0
        b = j // nmb

        pold = pold_ref[0]
        pt = pt_ref[0]
        d0 = pold[:, 0:1] - pt[0:1, :]
        d1 = pold[:, 1:2] - pt[1:2, :]
        d2c = pold[:, 2:3] - pt[2:3, :]
        d = d0 * d0 + d1 * d1 + d2c * d2c

        inf = jnp.float32(jnp.inf)
        m0 = jnp.min(d, axis=1, keepdims=True)
        dm = jnp.where(d == m0, inf, d)
        m1 = jnp.min(dm, axis=1, keepdims=True)
        dm = jnp.where(dm == m1, inf, dm)
        m2 = jnp.min(dm, axis=1, keepdims=True)

        w0 = 1.0 / jnp.maximum(m0, 1e-16)
        w1 = 1.0 / jnp.maximum(m1, 1e-16)
        w2 = 1.0 / jnp.maximum(m2, 1e-16)
        ws = w0 + w1 + w2
        zero = jnp.zeros_like(d)
        a = jnp.where(d == m0, w0 / ws,
                      jnp.where(d == m1, w1 / ws,
                                jnp.where(d == m2, w2 / ws, zero)))

        h_b = h_ref[pl.ds(b * N, N), :]
        interp = jax.lax.dot_general(
            a, h_b, (((1,), (0,)), ((), ())),
            preferred_element_type=jnp.float32)
        zlat_b = zlat_ref[pl.ds(j * MB, MB), :]
        lat = jnp.maximum(zlat_b * aff_ref[2:3, :] + aff_ref[3:4, :], 0.0)
        y_ref[0] = interp + lat


@functools.partial(jax.jit, static_argnames=())
def kernel(x, p, x_old, p_old, W_up, gamma_up, beta_up,
           W_lat, gamma_lat, beta_lat):
    B, N, Cin = x.shape
    M = p_old.shape[1]
    Cout = W_up.shape[0]
    nmb = M // MB
    np0 = (B * M) // MB1
    nsteps = np0 + B * nmb

    xf = x.reshape(B * N, Cin)
    xoldf = x_old.reshape(B * M, Cout)
    gb = jnp.stack([gamma_up, beta_up, gamma_lat, beta_lat], axis=0)
    p_t = jnp.transpose(p, (0, 2, 1))

    def _b_of(i):
        return jnp.where(i < np0, 0, (i - np0) // nmb)

    def _m_of(i):
        return jnp.where(i < np0, 0, (i - np0) % nmb)

    y = pl.pallas_call(
        functools.partial(_body, B, N, M, nmb, np0),
        grid=(nsteps,),
        in_specs=[
            pl.BlockSpec((B * N, Cin), lambda i: (0, 0)),
            pl.BlockSpec((MB1, Cout), lambda i: (jnp.where(i < np0, i, 0), 0)),
            pl.BlockSpec((Cout, Cin), lambda i: (0, 0)),
            pl.BlockSpec((Cout, Cout), lambda i: (0, 0)),
            pl.BlockSpec((1, 3, N), lambda i: (_b_of(i), 0, 0)),
            pl.BlockSpec((1, MB, 3), lambda i: (_b_of(i), _m_of(i), 0)),
            pl.BlockSpec((4, Cout), lambda i: (0, 0)),
        ],
        out_specs=pl.BlockSpec(
            (1, MB, Cout), lambda i: (_b_of(i), _m_of(i), 0)),
        out_shape=jax.ShapeDtypeStruct((B, M, Cout), jnp.float32),
        scratch_shapes=[
            pltpu.VMEM((B * N, Cout), jnp.float32),
            pltpu.VMEM((B * M, Cout), jnp.float32),
            pltpu.VMEM((4, Cout), jnp.float32),
            pltpu.VMEM((4, Cout), jnp.float32),
        ],
        compiler_params=pltpu.CompilerParams(
            dimension_semantics=("arbitrary",)),
    )(xf, xoldf, W_up, W_lat, p_t, p_old, gb)

    return (y, p_old)
